# baseline (device time: 115088 ns/iter reference)
import jax
import jax.numpy as jnp
from jax import lax
from jax.experimental import pallas as pl
from jax.experimental.pallas import tpu as pltpu

N_DEV = 16
B_LOC = 2
SQ = 128
D = 512
H_LOC = 8
DH = 64
SCALE = 0.125
ACHUNKS = 2
ACH_ROWS = SQ // ACHUNKS


def kernel(x, Wq, Wo, Wk, Wv):
    def contrib2(xr, xl, wq, wk, wv, wo, kmask, vmask):
        x2 = jnp.concatenate([xr, xl], axis=0)
        q2 = (jnp.dot(x2, wq, preferred_element_type=jnp.float32)
              * SCALE).astype(jnp.bfloat16)
        k2 = jnp.dot(x2, wk,
                     preferred_element_type=jnp.float32).astype(jnp.bfloat16)
        v2 = jnp.dot(x2, wv,
                     preferred_element_type=jnp.float32).astype(jnp.bfloat16)
        atts = []
        for half in range(2):
            rows = slice(half * SQ, (half + 1) * SQ)
            kT = k2[rows].T
            kbd = jnp.concatenate([kT] * H_LOC, axis=1) * kmask
            vbd = jnp.concatenate([v2[rows]] * H_LOC, axis=0) * vmask
            s = jnp.dot(q2[rows], kbd, preferred_element_type=jnp.float32)
            p = jnp.exp(s).astype(jnp.bfloat16)
            o = jnp.dot(p, vbd, preferred_element_type=jnp.float32)
            l = jnp.dot(p, vmask, preferred_element_type=jnp.float32)
            atts.append((o / l).astype(jnp.bfloat16))
        att2 = jnp.concatenate(atts, axis=0)
        out = jnp.dot(att2, wo, preferred_element_type=jnp.float32)
        return out[:SQ], out[SQ:]

    def body(x_ref, wq_ref, wo_ref, wk_ref, wv_ref, out_ref,
             xb_r, cb_r, ar_r, fin_r,
             xb_l, cb_l, ar_l, fin_l,
             xs_sems_r, xr_sems_r, as_sems_r, arx_sems_r,
             xs_sems_l, xr_sems_l, as_sems_l, arx_sems_l,
             fs_r, fr_r, fs_l, fr_l):
        me = lax.axis_index("i")
        q = lax.rem(me, 4)
        z = lax.div(me, 4)
        q_odd = lax.rem(q, 2) == 1
        right = jnp.where(
            q_odd,
            jnp.where(z < 3, me + 4, jnp.where(q == 1, 14, 12)),
            jnp.where(z > 0, me - 4, me + 1),
        )
        left = jnp.where(
            q_odd,
            jnp.where(z > 0, me - 4, me - 1),
            jnp.where(z < 3, me + 4, jnp.where(q == 2, 13, 15)),
        )

        barrier_sem = pltpu.get_barrier_semaphore()
        for nbr in (left, right):
            pl.semaphore_signal(
                barrier_sem, inc=1,
                device_id=(nbr,), device_id_type=pl.DeviceIdType.MESH,
            )
        pl.semaphore_wait(barrier_sem, 2)

        wq = wq_ref[...].astype(jnp.bfloat16)
        wk = wk_ref[...].astype(jnp.bfloat16)
        wv = wv_ref[...].astype(jnp.bfloat16)
        wo = wo_ref[...].astype(jnp.bfloat16)

        xb_r[0] = x_ref[0].astype(jnp.bfloat16)
        xb_l[0] = x_ref[1].astype(jnp.bfloat16)

        def x_fwd(xb, xs_sems, xr_sems, k, dst):
            return pltpu.make_async_remote_copy(
                src_ref=xb.at[k], dst_ref=xb.at[k + 1],
                send_sem=xs_sems.at[k], recv_sem=xr_sems.at[k],
                device_id=(dst,), device_id_type=pl.DeviceIdType.MESH,
            )

        def a_fwd(cb, ar, as_sems, arx_sems, k, c, dst):
            rows = slice(c * ACH_ROWS, (c + 1) * ACH_ROWS)
            return pltpu.make_async_remote_copy(
                src_ref=cb.at[k, rows], dst_ref=ar.at[k + 1, rows],
                send_sem=as_sems.at[k, c], recv_sem=arx_sems.at[k, c],
                device_id=(dst,), device_id_type=pl.DeviceIdType.MESH,
            )

        ci = lax.broadcasted_iota(jnp.int32, (D, SQ * H_LOC), 0)
        hi = lax.broadcasted_iota(jnp.int32, (D, SQ * H_LOC), 1)
        kmask = (lax.div(ci, DH) == lax.div(hi, SQ)).astype(jnp.bfloat16)
        ri = lax.broadcasted_iota(jnp.int32, (SQ * H_LOC, D), 0)
        cj = lax.broadcasted_iota(jnp.int32, (SQ * H_LOC, D), 1)
        vmask = (lax.div(ri, SQ) == lax.div(cj, DH)).astype(jnp.bfloat16)

        def compute_both(k):
            cr, cl = contrib2(xb_r[k], xb_l[k], wq, wk, wv, wo, kmask, vmask)
            return cr, cl

        def step(k, carry):
            @pl.when(k > 0)
            def _():
                for xb, xs, xr, dst in (
                    (xb_r, xs_sems_r, xr_sems_r, right),
                    (xb_l, xs_sems_l, xr_sems_l, left),
                ):
                    prev_x = x_fwd(xb, xs, xr, k - 1, dst)
                    prev_x.wait_recv()
                    prev_x.wait_send()

            cr, cl = compute_both(k)

            for c in range(ACHUNKS):
                rows = slice(c * ACH_ROWS, (c + 1) * ACH_ROWS)
                for cb, ar, a_sems, ax_sems, dst, cv in (
                    (cb_r, ar_r, as_sems_r, arx_sems_r, right, cr),
                    (cb_l, ar_l, as_sems_l, arx_sems_l, left, cl),
                ):
                    @pl.when(k > 0)
                    def _(cb=cb, ar=ar, a_sems=a_sems, ax_sems=ax_sems,
                          dst=dst, cv=cv, c=c, rows=rows):
                        prev_a = a_fwd(cb, ar, a_sems, ax_sems, k - 1, c, dst)
                        prev_a.wait_recv()
                        prev_a.wait_send()
                        cb[k, rows] = (cv[rows] + ar[k, rows].astype(
                            jnp.float32)).astype(jnp.bfloat16)

                    @pl.when(k == 0)
                    def _(cb=cb, cv=cv, rows=rows):
                        cb[0, rows] = cv[rows].astype(jnp.bfloat16)

                    a_fwd(cb, ar, a_sems, ax_sems, k, c, dst).start()

            x_fwd(xb_r, xs_sems_r, xr_sems_r, k, right).start()
            x_fwd(xb_l, xs_sems_l, xr_sems_l, k, left).start()
            return carry

        lax.fori_loop(0, N_DEV - 1, step, 0)

        k = N_DEV - 1
        for xb, xs, xr, dst in (
            (xb_r, xs_sems_r, xr_sems_r, right),
            (xb_l, xs_sems_l, xr_sems_l, left),
        ):
            prev_x = x_fwd(xb, xs, xr, k - 1, dst)
            prev_x.wait_recv()
            prev_x.wait_send()
        cr, cl = compute_both(k)
        fin_rdmas = []
        for c in range(ACHUNKS):
            rows = slice(c * ACH_ROWS, (c + 1) * ACH_ROWS)
            for cb, ar, a_sems, ax_sems, fin, fs, fr, dst, cv in (
                (cb_r, ar_r, as_sems_r, arx_sems_r, fin_r, fs_r, fr_r,
                 right, cr),
                (cb_l, ar_l, as_sems_l, arx_sems_l, fin_l, fs_l, fr_l,
                 left, cl),
            ):
                prev_a = a_fwd(cb, ar, a_sems, ax_sems, k - 1, c, dst)
                prev_a.wait_recv()
                prev_a.wait_send()
                cb[k, rows] = (cv[rows] + ar[k, rows].astype(
                    jnp.float32)).astype(jnp.bfloat16)
                fin_rdma = pltpu.make_async_remote_copy(
                    src_ref=cb.at[k, rows], dst_ref=fin.at[rows],
                    send_sem=fs.at[c], recv_sem=fr.at[c],
                    device_id=(dst,), device_id_type=pl.DeviceIdType.MESH,
                )
                fin_rdma.start()
                fin_rdmas.append(fin_rdma)
        for fin_rdma in fin_rdmas:
            fin_rdma.wait()

        out_ref[0] = fin_r[...].astype(jnp.float32)
        out_ref[1] = fin_l[...].astype(jnp.float32)

    ring_scratch = [
        pltpu.VMEM((N_DEV, SQ, D), jnp.bfloat16),
        pltpu.VMEM((N_DEV, SQ, D), jnp.bfloat16),
        pltpu.VMEM((N_DEV, SQ, D), jnp.bfloat16),
        pltpu.VMEM((SQ, D), jnp.bfloat16),
    ]
    ring_sems = [
        pltpu.SemaphoreType.DMA((N_DEV - 1,)),
        pltpu.SemaphoreType.DMA((N_DEV - 1,)),
        pltpu.SemaphoreType.DMA((N_DEV - 1, ACHUNKS)),
        pltpu.SemaphoreType.DMA((N_DEV - 1, ACHUNKS)),
    ]

    return pl.pallas_call(
        body,
        out_shape=jax.ShapeDtypeStruct((B_LOC, SQ, D), jnp.float32),
        in_specs=[pl.BlockSpec(memory_space=pltpu.VMEM)] * 5,
        out_specs=pl.BlockSpec(memory_space=pltpu.VMEM),
        scratch_shapes=(
            ring_scratch + ring_scratch
            + ring_sems + ring_sems
            + [pltpu.SemaphoreType.DMA((ACHUNKS,))] * 4
        ),
        compiler_params=pltpu.CompilerParams(collective_id=0),
    )(x, Wq, Wo, Wk, Wv)


# device time: 65638 ns/iter; 1.7534x vs baseline; 1.7534x over previous
import jax
import jax.numpy as jnp
from jax import lax
from jax.experimental import pallas as pl
from jax.experimental.pallas import tpu as pltpu

N_DEV = 16
B_LOC = 2
SQ = 128
D = 512
H_LOC = 8
DH = 64
SCALE = 0.125
ACHUNKS = 2
ACH_ROWS = SQ // ACHUNKS


def kernel(x, Wq, Wo, Wk, Wv):
    def contrib2(xr, xl, wq, wk, wv, wo, kmask, vmask):
        x2 = jnp.concatenate([xr, xl], axis=0)
        q2 = (jnp.dot(x2, wq, preferred_element_type=jnp.float32)
              * SCALE).astype(jnp.bfloat16)
        k2 = jnp.dot(x2, wk,
                     preferred_element_type=jnp.float32).astype(jnp.bfloat16)
        v2 = jnp.dot(x2, wv,
                     preferred_element_type=jnp.float32).astype(jnp.bfloat16)
        atts = []
        for half in range(2):
            rows = slice(half * SQ, (half + 1) * SQ)
            kT = k2[rows].T
            kbd = jnp.concatenate([kT] * H_LOC, axis=1) * kmask
            vbd = jnp.concatenate([v2[rows]] * H_LOC, axis=0) * vmask
            s = jnp.dot(q2[rows], kbd, preferred_element_type=jnp.float32)
            p = jnp.exp(s).astype(jnp.bfloat16)
            o = jnp.dot(p, vbd, preferred_element_type=jnp.float32)
            l = jnp.dot(p, vmask, preferred_element_type=jnp.float32)
            atts.append((o / l).astype(jnp.bfloat16))
        att2 = jnp.concatenate(atts, axis=0)
        out = jnp.dot(att2, wo, preferred_element_type=jnp.float32)
        return out[:SQ], out[SQ:]

    def body(x_ref, wq_ref, wo_ref, wk_ref, wv_ref, out_ref,
             xb_r, cb_r, ar_r, fin_r,
             xb_l, cb_l, ar_l, fin_l,
             xs_sems_r, xr_sems_r, as_sems_r, arx_sems_r,
             xs_sems_l, xr_sems_l, as_sems_l, arx_sems_l,
             fs_r, fr_r, fs_l, fr_l):
        me = lax.axis_index("i")
        q = lax.rem(me, 4)
        z = lax.div(me, 4)
        q_odd = lax.rem(q, 2) == 1
        right = jnp.where(
            q_odd,
            jnp.where(z < 3, me + 4, jnp.where(q == 1, 14, 12)),
            jnp.where(z > 0, me - 4, me + 1),
        )
        left = jnp.where(
            q_odd,
            jnp.where(z > 0, me - 4, me - 1),
            jnp.where(z < 3, me + 4, jnp.where(q == 2, 13, 15)),
        )

        barrier_sem = pltpu.get_barrier_semaphore()
        for nbr in (left, right):
            pl.semaphore_signal(
                barrier_sem, inc=1,
                device_id=(nbr,), device_id_type=pl.DeviceIdType.MESH,
            )
        pl.semaphore_wait(barrier_sem, 2)

        wq = wq_ref[...].astype(jnp.bfloat16)
        wk = wk_ref[...].astype(jnp.bfloat16)
        wv = wv_ref[...].astype(jnp.bfloat16)
        wo = wo_ref[...].astype(jnp.bfloat16)

        xb_r[0] = x_ref[0].astype(jnp.bfloat16)
        xb_l[0] = x_ref[1].astype(jnp.bfloat16)

        def x_fwd(xb, xs_sems, xr_sems, k, dst):
            return pltpu.make_async_remote_copy(
                src_ref=xb.at[k], dst_ref=xb.at[k + 1],
                send_sem=xs_sems.at[k], recv_sem=xr_sems.at[k],
                device_id=(dst,), device_id_type=pl.DeviceIdType.MESH,
            )

        def a_fwd(cb, ar, as_sems, arx_sems, k, c, dst):
            rows = slice(c * ACH_ROWS, (c + 1) * ACH_ROWS)
            return pltpu.make_async_remote_copy(
                src_ref=cb.at[k, rows], dst_ref=ar.at[k + 1, rows],
                send_sem=as_sems.at[k, c], recv_sem=arx_sems.at[k, c],
                device_id=(dst,), device_id_type=pl.DeviceIdType.MESH,
            )

        ci = lax.broadcasted_iota(jnp.int32, (D, SQ * H_LOC), 0)
        hi = lax.broadcasted_iota(jnp.int32, (D, SQ * H_LOC), 1)
        kmask = (lax.div(ci, DH) == lax.div(hi, SQ)).astype(jnp.bfloat16)
        ri = lax.broadcasted_iota(jnp.int32, (SQ * H_LOC, D), 0)
        cj = lax.broadcasted_iota(jnp.int32, (SQ * H_LOC, D), 1)
        vmask = (lax.div(ri, SQ) == lax.div(cj, DH)).astype(jnp.bfloat16)

        def compute_both(k):
            cr, cl = contrib2(xb_r[k], xb_l[k], wq, wk, wv, wo, kmask, vmask)
            return cr, cl

        def step(k, carry):
            @pl.when(k > 0)
            def _():
                for xb, xs, xr, dst in (
                    (xb_r, xs_sems_r, xr_sems_r, right),
                    (xb_l, xs_sems_l, xr_sems_l, left),
                ):
                    prev_x = x_fwd(xb, xs, xr, k - 1, dst)
                    prev_x.wait_recv()
                    prev_x.wait_send()

            x_fwd(xb_r, xs_sems_r, xr_sems_r, k, right).start()
            x_fwd(xb_l, xs_sems_l, xr_sems_l, k, left).start()
            cr, cl = compute_both(k)

            for c in range(ACHUNKS):
                rows = slice(c * ACH_ROWS, (c + 1) * ACH_ROWS)
                for cb, ar, a_sems, ax_sems, dst, cv in (
                    (cb_r, ar_r, as_sems_r, arx_sems_r, right, cr),
                    (cb_l, ar_l, as_sems_l, arx_sems_l, left, cl),
                ):
                    @pl.when(k > 0)
                    def _(cb=cb, ar=ar, a_sems=a_sems, ax_sems=ax_sems,
                          dst=dst, cv=cv, c=c, rows=rows):
                        prev_a = a_fwd(cb, ar, a_sems, ax_sems, k - 1, c, dst)
                        prev_a.wait_recv()
                        prev_a.wait_send()
                        cb[k, rows] = (cv[rows] + ar[k, rows].astype(
                            jnp.float32)).astype(jnp.bfloat16)

                    @pl.when(k == 0)
                    def _(cb=cb, cv=cv, rows=rows):
                        cb[0, rows] = cv[rows].astype(jnp.bfloat16)

                    a_fwd(cb, ar, a_sems, ax_sems, k, c, dst).start()
            return carry

        lax.fori_loop(0, N_DEV - 1, step, 0)

        k = N_DEV - 1
        for xb, xs, xr, dst in (
            (xb_r, xs_sems_r, xr_sems_r, right),
            (xb_l, xs_sems_l, xr_sems_l, left),
        ):
            prev_x = x_fwd(xb, xs, xr, k - 1, dst)
            prev_x.wait_recv()
            prev_x.wait_send()
        cr, cl = compute_both(k)
        fin_rdmas = []
        for c in range(ACHUNKS):
            rows = slice(c * ACH_ROWS, (c + 1) * ACH_ROWS)
            for cb, ar, a_sems, ax_sems, fin, fs, fr, dst, cv in (
                (cb_r, ar_r, as_sems_r, arx_sems_r, fin_r, fs_r, fr_r,
                 right, cr),
                (cb_l, ar_l, as_sems_l, arx_sems_l, fin_l, fs_l, fr_l,
                 left, cl),
            ):
                prev_a = a_fwd(cb, ar, a_sems, ax_sems, k - 1, c, dst)
                prev_a.wait_recv()
                prev_a.wait_send()
                cb[k, rows] = (cv[rows] + ar[k, rows].astype(
                    jnp.float32)).astype(jnp.bfloat16)
                fin_rdma = pltpu.make_async_remote_copy(
                    src_ref=cb.at[k, rows], dst_ref=fin.at[rows],
                    send_sem=fs.at[c], recv_sem=fr.at[c],
                    device_id=(dst,), device_id_type=pl.DeviceIdType.MESH,
                )
                fin_rdma.start()
                fin_rdmas.append(fin_rdma)
        for fin_rdma in fin_rdmas:
            fin_rdma.wait()

        out_ref[0] = fin_r[...].astype(jnp.float32)
        out_ref[1] = fin_l[...].astype(jnp.float32)

    ring_scratch = [
        pltpu.VMEM((N_DEV, SQ, D), jnp.bfloat16),
        pltpu.VMEM((N_DEV, SQ, D), jnp.bfloat16),
        pltpu.VMEM((N_DEV, SQ, D), jnp.bfloat16),
        pltpu.VMEM((SQ, D), jnp.bfloat16),
    ]
    ring_sems = [
        pltpu.SemaphoreType.DMA((N_DEV - 1,)),
        pltpu.SemaphoreType.DMA((N_DEV - 1,)),
        pltpu.SemaphoreType.DMA((N_DEV - 1, ACHUNKS)),
        pltpu.SemaphoreType.DMA((N_DEV - 1, ACHUNKS)),
    ]

    return pl.pallas_call(
        body,
        out_shape=jax.ShapeDtypeStruct((B_LOC, SQ, D), jnp.float32),
        in_specs=[pl.BlockSpec(memory_space=pltpu.VMEM)] * 5,
        out_specs=pl.BlockSpec(memory_space=pltpu.VMEM),
        scratch_shapes=(
            ring_scratch + ring_scratch
            + ring_sems + ring_sems
            + [pltpu.SemaphoreType.DMA((ACHUNKS,))] * 4
        ),
        compiler_params=pltpu.CompilerParams(collective_id=0),
    )(x, Wq, Wo, Wk, Wv)
